# baseline (speedup 1.0000x reference)
import functools

import jax
import jax.numpy as jnp
from jax import lax
from jax.experimental import pallas as pl
from jax.experimental.pallas import tpu as pltpu
from jax.experimental.pallas import tpu_sc as plsc

N_NODES = 100000
N_EDGES = 3200000

NC = 2
NS = 16
NW = NC * NS

NT = 100352
TRASH = N_NODES

CHUNK = 2048
SUB = 16
NITER = 49
EPW = CHUNK * NITER
EP = EPW * NW
EP_ROWS = EP // 128
WROWS = EPW // 128
RPT = NT // NS
SCH = 784
SIT = RPT // SCH
SVR = SCH * 8 // 16

_mesh = plsc.VectorSubcoreMesh(core_axis_name="c", subcore_axis_name="s")


def _frsqrt(x):
    i = plsc.bitcast(x, jnp.int32)
    i = jnp.int32(0x5F3759DF) - lax.shift_right_arithmetic(i, 1)
    y = plsc.bitcast(i, jnp.float32)
    for _ in range(3):
        y = y * (1.5 - 0.5 * x * y * y)
    return y


@functools.partial(
    pl.kernel,
    out_type=jax.ShapeDtypeStruct((NC, NT), jnp.float32),
    mesh=_mesh,
    scratch_types=[
        pltpu.VMEM((SUB, 128), jnp.int32),
        pltpu.VMEM((128,), jnp.float32),
        pltpu.VMEM_SHARED((NT,), jnp.float32),
        pltpu.SemaphoreType.DMA,
    ],
)
def _sc_degree(dst_hbm, ones_hbm, zeros_hbm, degp_hbm, idx_v, ones_v, acc_sh, sem):
    c = lax.axis_index("c")
    s = lax.axis_index("s")
    w = s * NC + c
    base = s * RPT
    pltpu.sync_copy(zeros_hbm.at[pl.ds(base, RPT)], acc_sh.at[pl.ds(base, RPT)])
    pltpu.sync_copy(ones_hbm, ones_v)
    plsc.subcore_barrier()

    def chunk(g, carry):
        row = w * WROWS + g * SUB
        pltpu.async_copy(dst_hbm.at[pl.ds(row, SUB)], idx_v, sem).wait()
        for j in range(SUB):
            pltpu.sync_copy(ones_v, acc_sh.at[idx_v.at[j]], add=True)
        return carry

    lax.fori_loop(0, NITER, chunk, 0)
    plsc.subcore_barrier()
    pltpu.sync_copy(acc_sh.at[pl.ds(base, RPT)],
                    degp_hbm.at[c, pl.ds(base, RPT)])


def _make_layer(edge_split):
    if edge_split:
        ait, ch, sb = 98, 1024, 8
        arows = EP_ROWS // NW
        w_rows = NT
    else:
        ait, ch, sb = 98, 2048, 16
        arows = EP_ROWS // NS
        w_rows = NC * NT

    @functools.partial(
        pl.kernel,
        out_type=jax.ShapeDtypeStruct((NC * NT, 8), jnp.float32),
        mesh=_mesh,
        compiler_params=pltpu.CompilerParams(use_tc_tiling_on_sc=False,
                                             needs_layout_passes=False),
        scratch_types=[
            pltpu.VMEM((RPT,), jnp.float32),
            pltpu.VMEM((RPT,), jnp.float32),
            pltpu.VMEM((RPT,), jnp.float32),
            pltpu.VMEM((SCH, 8), jnp.float32),
            pltpu.VMEM((sb, 128), jnp.int32),
            pltpu.VMEM((sb, 128), jnp.int32),
            pltpu.VMEM((sb, 128), jnp.int32),
            pltpu.VMEM((sb, 128), jnp.int32),
            pltpu.VMEM((ch, 8), jnp.float32),
            pltpu.VMEM((ch, 8), jnp.float32),
            pltpu.VMEM_SHARED((NT, 8), jnp.float32),
            pltpu.SemaphoreType.DMA,
            pltpu.SemaphoreType.DMA,
            pltpu.SemaphoreType.DMA,
        ],
    )
    def _sc_layer(src2_hbm, dst_hbm, w_hbm, degp_hbm, zeros_hbm, z_hbm,
                  dp0, dp1, dinv_v, wbuf, sidx0, didx0, sidx1, didx1,
                  rows0, rows1, acc_sh, gsem, ssem, isem):
        c = lax.axis_index("c")
        s = lax.axis_index("s")
        t0 = s * RPT
        ji = lax.iota(jnp.int32, 16)
        rp2 = ji // 8
        colp = ji % 8

        pltpu.sync_copy(degp_hbm.at[0, pl.ds(t0, RPT)], dp0)
        pltpu.sync_copy(degp_hbm.at[1, pl.ds(t0, RPT)], dp1)

        def dinv_step(k, carry):
            sl = pl.ds(k * 16, 16)
            dinv_v[sl] = _frsqrt(dp0[sl] + dp1[sl] + 1.0)
            return carry

        lax.fori_loop(0, RPT // 16, dinv_step, 0)

        def stage_a(i, carry):
            start = t0 + i * SCH
            woff = start if edge_split else c * NT + start
            pltpu.sync_copy(w_hbm.at[pl.ds(woff, SCH)], wbuf)

            def scale(v, carry2):
                wr = rp2 + 2 * v
                w16 = plsc.load_gather(wbuf, [wr, colp])
                dv = plsc.load_gather(dinv_v, [rp2 + (i * SCH + 2 * v)])
                plsc.store_scatter(wbuf, [wr, colp], w16 * dv)
                return carry2

            lax.fori_loop(0, SVR, scale, 0)
            pltpu.sync_copy(wbuf, z_hbm.at[pl.ds(c * NT + start, SCH)])
            if edge_split:
                @pl.when(c == 0)
                def _():
                    pltpu.sync_copy(wbuf, acc_sh.at[pl.ds(start, SCH)])

                @pl.when(c != 0)
                def _():
                    pltpu.sync_copy(zeros_hbm.at[pl.ds(start, SCH)],
                                    acc_sh.at[pl.ds(start, SCH)])
            else:
                pltpu.sync_copy(wbuf, acc_sh.at[pl.ds(start, SCH)])
            return carry

        lax.fori_loop(0, SIT, stage_a, 0)
        plsc.subcore_barrier()

        def row0(k):
            if edge_split:
                return (c * NS + s) * arows + k * sb
            return s * arows + k * sb

        def load_idx(k, sbuf, dbuf):
            h1 = pltpu.async_copy(src2_hbm.at[c, pl.ds(row0(k), sb)],
                                  sbuf, isem)
            h2 = pltpu.async_copy(dst_hbm.at[pl.ds(row0(k), sb)], dbuf, isem)
            return h1, h2

        def fire_gathers(sbuf, rbuf):
            return [pltpu.async_copy(z_hbm.at[sbuf.at[j]],
                                     rbuf.at[pl.ds(j * 128, 128)], gsem)
                    for j in range(sb)]

        def fire_scatters(dbuf, rbuf):
            return [pltpu.async_copy(rbuf.at[pl.ds(j * 128, 128)],
                                     acc_sh.at[dbuf.at[j]], ssem, add=True)
                    for j in range(sb)]

        def drain_gathers(rbuf):
            pltpu.make_async_copy(w_hbm.at[pl.ds(0, ch)], rbuf, gsem).wait()

        h1, h2 = load_idx(0, sidx0, didx0)
        h1.wait()
        h2.wait()
        fire_gathers(sidx0, rows0)

        def pair(g, carry):
            a = 2 * g
            l1, l2 = load_idx(a + 1, sidx1, didx1)
            drain_gathers(rows0)
            sa = fire_scatters(didx0, rows0)
            l1.wait()
            l2.wait()
            gb = fire_gathers(sidx1, rows1)
            for h in sa:
                h.wait()

            @pl.when(g < ait // 2 - 1)
            def _():
                n1, n2 = load_idx(a + 2, sidx0, didx0)
                n1.wait()
                n2.wait()
                fire_gathers(sidx0, rows0)

            for h in gb:
                h.wait()
            sb_h = fire_scatters(didx1, rows1)
            for h in sb_h:
                h.wait()
            return carry

        lax.fori_loop(0, ait // 2, pair, 0)
        plsc.subcore_barrier()

        def stage_c(i, carry):
            start = t0 + i * SCH
            pltpu.sync_copy(acc_sh.at[pl.ds(start, SCH)], wbuf)

            def scale(v, carry2):
                wr = rp2 + 2 * v
                a16 = plsc.load_gather(wbuf, [wr, colp])
                dv = plsc.load_gather(dinv_v, [rp2 + (i * SCH + 2 * v)])
                plsc.store_scatter(wbuf, [wr, colp], a16 * dv)
                return carry2

            lax.fori_loop(0, SVR, scale, 0)
            pltpu.sync_copy(wbuf, z_hbm.at[pl.ds(c * NT + start, SCH)])
            return carry

        lax.fori_loop(0, SIT, stage_c, 0)

    return _sc_layer


_sc_layer1 = _make_layer(edge_split=False)
_sc_layer2 = _make_layer(edge_split=True)


_WROW = NT // 16
_TB = 448


def _dense1_body(xr_ref, m0_ref, m1_ref, outw_ref):
    xb = xr_ref[...]
    outw_ref[0] = jnp.dot(xb, m0_ref[...], preferred_element_type=jnp.float32)
    outw_ref[1] = jnp.dot(xb, m1_ref[...], preferred_element_type=jnp.float32)


def _dense2_body(zw_ref, b1w_ref, k00_ref, k10_ref, outw_ref):
    h0 = jnp.maximum(zw_ref[0] + b1w_ref[0], 0.0)
    h1 = jnp.maximum(zw_ref[1] + b1w_ref[1], 0.0)
    outw_ref[...] = (jnp.dot(h0, k00_ref[...], preferred_element_type=jnp.float32)
                     + jnp.dot(h1, k10_ref[...], preferred_element_type=jnp.float32))


def _tc_dense1(xr, m0, m1):
    return pl.pallas_call(
        _dense1_body,
        grid=(_WROW // _TB,),
        in_specs=[
            pl.BlockSpec((_TB, 48), lambda i: (i, 0)),
            pl.BlockSpec((48, 128), lambda i: (0, 0)),
            pl.BlockSpec((48, 128), lambda i: (0, 0)),
        ],
        out_specs=pl.BlockSpec((2, _TB, 128), lambda i: (0, i, 0)),
        out_shape=jax.ShapeDtypeStruct((2, _WROW, 128), jnp.float32),
    )(xr, m0, m1)


def _tc_dense2(zw, b1w, k00, k10):
    return pl.pallas_call(
        _dense2_body,
        grid=(_WROW // _TB,),
        in_specs=[
            pl.BlockSpec((2, _TB, 128), lambda i: (0, i, 0)),
            pl.BlockSpec((2, 128), lambda i: (0, 0)),
            pl.BlockSpec((128, 128), lambda i: (0, 0)),
            pl.BlockSpec((128, 128), lambda i: (0, 0)),
        ],
        out_specs=pl.BlockSpec((_TB, 128), lambda i: (i, 0)),
        out_shape=jax.ShapeDtypeStruct((_WROW, 128), jnp.float32),
    )(zw, b1w, k00, k10)


def kernel(x, edge_index, W1, b1, W2, b2):
    f32 = jnp.float32
    ei = edge_index.astype(jnp.int32)
    pad = jnp.full((EP - N_EDGES,), TRASH, jnp.int32)
    padd = (jnp.arange(EP - N_EDGES, dtype=jnp.int32) % (NT - N_NODES)) + TRASH
    src_p = jnp.concatenate([ei[0], pad])
    dst_p = jnp.concatenate([ei[1], padd])
    src2 = jnp.stack([src_p, src_p + NT]).reshape(NC, EP_ROWS, 128)
    dst2d = dst_p.reshape(EP_ROWS, 128)

    eye16 = jnp.eye(16, dtype=f32)
    W1f = W1.astype(f32)
    W2p = jnp.pad(W2.astype(f32), ((0, 0), (0, 1)))
    m0 = jnp.kron(eye16, W1f[:, :8])
    m1 = jnp.kron(eye16, W1f[:, 8:])
    k00 = jnp.kron(eye16, W2p[:8, :])
    k10 = jnp.kron(eye16, W2p[8:, :])
    b1w = jnp.tile(b1.astype(f32).reshape(2, 8), (1, 16))

    ones128 = jnp.ones((128,), f32)
    zerosNT = jnp.zeros((NT,), f32)
    zeros8 = jnp.zeros((NT, 8), f32)

    xr = x.reshape(N_NODES * 3 // 48, 48)

    degp = _sc_degree(dst2d, ones128, zerosNT)

    w1t = _tc_dense1(xr, m0, m1).reshape(NC * NT, 8)
    z1 = _sc_layer1(src2, dst2d, w1t, degp, zeros8)

    w2t = _tc_dense2(z1.reshape(NC, _WROW, 128), b1w, k00, k10)
    z2 = _sc_layer2(src2, dst2d, w2t.reshape(NT, 8), degp, zeros8)

    b2w = jnp.tile(jnp.pad(b2.astype(f32), (0, 1)), 16)
    z2w = z2.reshape(NC, _WROW, 128)
    outw = z2w[0] + z2w[1] + b2w[None, :]
    return outw.reshape(NT, 8)[:N_NODES, :7]


# speedup vs baseline: 1.0616x; 1.0616x over previous
import functools

import jax
import jax.numpy as jnp
from jax import lax
from jax.experimental import pallas as pl
from jax.experimental.pallas import tpu as pltpu
from jax.experimental.pallas import tpu_sc as plsc

N_NODES = 100000
N_EDGES = 3200000

NC = 2
NS = 16
NW = NC * NS

NT = 100352
TRASH = N_NODES

CHUNK = 2048
SUB = 16
NITER = 49
EPW = CHUNK * NITER
EP = EPW * NW
EP_ROWS = EP // 128
WROWS = EPW // 128
RPT = NT // NS
SCH = 784
SIT = RPT // SCH
SVR = SCH * 8 // 16

_mesh = plsc.VectorSubcoreMesh(core_axis_name="c", subcore_axis_name="s")


def _frsqrt(x):
    i = plsc.bitcast(x, jnp.int32)
    i = jnp.int32(0x5F3759DF) - lax.shift_right_arithmetic(i, 1)
    y = plsc.bitcast(i, jnp.float32)
    for _ in range(3):
        y = y * (1.5 - 0.5 * x * y * y)
    return y


@functools.partial(
    pl.kernel,
    out_type=jax.ShapeDtypeStruct((NC, NT), jnp.float32),
    mesh=_mesh,
    scratch_types=[
        pltpu.VMEM((SUB, 128), jnp.int32),
        pltpu.VMEM((128,), jnp.float32),
        pltpu.VMEM_SHARED((NT,), jnp.float32),
        pltpu.SemaphoreType.DMA,
    ],
)
def _sc_degree(dst_hbm, ones_hbm, zeros_hbm, degp_hbm, idx_v, ones_v, acc_sh, sem):
    c = lax.axis_index("c")
    s = lax.axis_index("s")
    w = s * NC + c
    base = s * RPT
    pltpu.sync_copy(zeros_hbm.at[pl.ds(base, RPT)], acc_sh.at[pl.ds(base, RPT)])
    pltpu.sync_copy(ones_hbm, ones_v)
    plsc.subcore_barrier()

    def chunk(g, carry):
        row = w * WROWS + g * SUB
        pltpu.async_copy(dst_hbm.at[pl.ds(row, SUB)], idx_v, sem).wait()
        for j in range(SUB):
            pltpu.sync_copy(ones_v, acc_sh.at[idx_v.at[j]], add=True)
        return carry

    lax.fori_loop(0, NITER, chunk, 0)
    plsc.subcore_barrier()
    pltpu.sync_copy(acc_sh.at[pl.ds(base, RPT)],
                    degp_hbm.at[c, pl.ds(base, RPT)])


def _make_layer(edge_split):
    if edge_split:
        ait, ch, sb = 98, 1024, 8
        arows = EP_ROWS // NW
        w_rows = NT
    else:
        ait, ch, sb = 98, 2048, 16
        arows = EP_ROWS // NS
        w_rows = NC * NT

    @functools.partial(
        pl.kernel,
        out_type=jax.ShapeDtypeStruct((NC * NT, 8), jnp.float32),
        mesh=_mesh,
        compiler_params=pltpu.CompilerParams(use_tc_tiling_on_sc=False,
                                             needs_layout_passes=False),
        scratch_types=[
            pltpu.VMEM((RPT,), jnp.float32),
            pltpu.VMEM((RPT,), jnp.float32),
            pltpu.VMEM((RPT,), jnp.float32),
            pltpu.VMEM((SCH, 8), jnp.float32),
            pltpu.VMEM((sb, 128), jnp.int32),
            pltpu.VMEM((sb, 128), jnp.int32),
            pltpu.VMEM((sb, 128), jnp.int32),
            pltpu.VMEM((sb, 128), jnp.int32),
            pltpu.VMEM((ch, 8), jnp.float32),
            pltpu.VMEM((ch, 8), jnp.float32),
            pltpu.VMEM_SHARED((NT, 8), jnp.float32),
            pltpu.SemaphoreType.DMA,
            pltpu.SemaphoreType.DMA,
            pltpu.SemaphoreType.DMA,
        ],
    )
    def _sc_layer(src2_hbm, dst_hbm, w_hbm, degp_hbm, zeros_hbm, z_hbm,
                  dp0, dp1, dinv_v, wbuf, sidx0, didx0, sidx1, didx1,
                  rows0, rows1, acc_sh, gsem, ssem, isem):
        c = lax.axis_index("c")
        s = lax.axis_index("s")
        t0 = s * RPT
        ji = lax.iota(jnp.int32, 16)
        rp2 = ji // 8
        colp = ji % 8

        pltpu.sync_copy(degp_hbm.at[0, pl.ds(t0, RPT)], dp0)
        pltpu.sync_copy(degp_hbm.at[1, pl.ds(t0, RPT)], dp1)

        def dinv_step(k, carry):
            sl = pl.ds(k * 16, 16)
            dinv_v[sl] = _frsqrt(dp0[sl] + dp1[sl] + 1.0)
            return carry

        lax.fori_loop(0, RPT // 16, dinv_step, 0)

        def stage_a(i, carry):
            start = t0 + i * SCH
            woff = start if edge_split else c * NT + start
            pltpu.sync_copy(w_hbm.at[pl.ds(woff, SCH)], wbuf)

            def scale(v, carry2):
                wr = rp2 + 2 * v
                w16 = plsc.load_gather(wbuf, [wr, colp])
                dv = plsc.load_gather(dinv_v, [rp2 + (i * SCH + 2 * v)])
                plsc.store_scatter(wbuf, [wr, colp], w16 * dv)
                return carry2

            lax.fori_loop(0, SVR, scale, 0)
            pltpu.sync_copy(wbuf, z_hbm.at[pl.ds(c * NT + start, SCH)])
            if edge_split:
                @pl.when(c == 0)
                def _():
                    pltpu.sync_copy(wbuf, acc_sh.at[pl.ds(start, SCH)])

                @pl.when(c != 0)
                def _():
                    pltpu.sync_copy(zeros_hbm.at[pl.ds(start, SCH)],
                                    acc_sh.at[pl.ds(start, SCH)])
            else:
                pltpu.sync_copy(wbuf, acc_sh.at[pl.ds(start, SCH)])
            return carry

        lax.fori_loop(0, SIT, stage_a, 0)
        plsc.subcore_barrier()

        def row0(k):
            if edge_split:
                return (c * NS + s) * arows + k * sb
            return s * arows + k * sb

        def load_idx(k, sbuf, dbuf):
            h1 = pltpu.async_copy(src2_hbm.at[c, pl.ds(row0(k), sb)],
                                  sbuf, isem)
            h2 = pltpu.async_copy(dst_hbm.at[pl.ds(row0(k), sb)], dbuf, isem)
            return h1, h2

        def fire_gathers(sbuf, rbuf):
            return [pltpu.async_copy(z_hbm.at[sbuf.at[j]],
                                     rbuf.at[pl.ds(j * 128, 128)], gsem)
                    for j in range(sb)]

        def fire_scatters(dbuf, rbuf):
            return [pltpu.async_copy(rbuf.at[pl.ds(j * 128, 128)],
                                     acc_sh.at[dbuf.at[j]], ssem, add=True)
                    for j in range(sb)]

        def drain_gathers(rbuf):
            pltpu.make_async_copy(w_hbm.at[pl.ds(0, ch)], rbuf, gsem).wait()

        h1, h2 = load_idx(0, sidx0, didx0)
        h1.wait()
        h2.wait()
        fire_gathers(sidx0, rows0)

        def pair(g, carry):
            a = 2 * g
            l1, l2 = load_idx(a + 1, sidx1, didx1)
            drain_gathers(rows0)
            sa = fire_scatters(didx0, rows0)
            l1.wait()
            l2.wait()
            gb = fire_gathers(sidx1, rows1)
            for h in sa:
                h.wait()

            @pl.when(g < ait // 2 - 1)
            def _():
                n1, n2 = load_idx(a + 2, sidx0, didx0)
                n1.wait()
                n2.wait()
                fire_gathers(sidx0, rows0)

            for h in gb:
                h.wait()
            sb_h = fire_scatters(didx1, rows1)
            for h in sb_h:
                h.wait()
            return carry

        lax.fori_loop(0, ait // 2, pair, 0)
        plsc.subcore_barrier()

        def stage_c(i, carry):
            start = t0 + i * SCH
            pltpu.sync_copy(acc_sh.at[pl.ds(start, SCH)], wbuf)

            def scale(v, carry2):
                wr = rp2 + 2 * v
                a16 = plsc.load_gather(wbuf, [wr, colp])
                dv = plsc.load_gather(dinv_v, [rp2 + (i * SCH + 2 * v)])
                plsc.store_scatter(wbuf, [wr, colp], a16 * dv)
                return carry2

            lax.fori_loop(0, SVR, scale, 0)
            pltpu.sync_copy(wbuf, z_hbm.at[pl.ds(c * NT + start, SCH)])
            return carry

        lax.fori_loop(0, SIT, stage_c, 0)

    return _sc_layer


_sc_layer1 = _make_layer(edge_split=False)
_sc_layer2 = _make_layer(edge_split=True)


_WROW = NT // 16
_TB = 448


def _dense1_body(xr_ref, m0_ref, m1_ref, outw_ref):
    xb = xr_ref[...]
    outw_ref[0] = jnp.dot(xb, m0_ref[...], preferred_element_type=jnp.float32)
    outw_ref[1] = jnp.dot(xb, m1_ref[...], preferred_element_type=jnp.float32)


def _dense2_body(zw_ref, b1w_ref, k00_ref, k10_ref, outw_ref):
    h0 = jnp.maximum(zw_ref[0] + b1w_ref[0], 0.0)
    h1 = jnp.maximum(zw_ref[1] + b1w_ref[1], 0.0)
    outw_ref[...] = (jnp.dot(h0, k00_ref[...], preferred_element_type=jnp.float32)
                     + jnp.dot(h1, k10_ref[...], preferred_element_type=jnp.float32))


def _tc_dense1(xr, m0, m1):
    return pl.pallas_call(
        _dense1_body,
        grid=(_WROW // _TB,),
        in_specs=[
            pl.BlockSpec((_TB, 48), lambda i: (i, 0)),
            pl.BlockSpec((48, 128), lambda i: (0, 0)),
            pl.BlockSpec((48, 128), lambda i: (0, 0)),
        ],
        out_specs=pl.BlockSpec((2, _TB, 128), lambda i: (0, i, 0)),
        out_shape=jax.ShapeDtypeStruct((2, _WROW, 128), jnp.float32),
    )(xr, m0, m1)


def _tc_dense2(zw, b1w, k00, k10):
    return pl.pallas_call(
        _dense2_body,
        grid=(_WROW // _TB,),
        in_specs=[
            pl.BlockSpec((2, _TB, 128), lambda i: (0, i, 0)),
            pl.BlockSpec((2, 128), lambda i: (0, 0)),
            pl.BlockSpec((128, 128), lambda i: (0, 0)),
            pl.BlockSpec((128, 128), lambda i: (0, 0)),
        ],
        out_specs=pl.BlockSpec((_TB, 128), lambda i: (i, 0)),
        out_shape=jax.ShapeDtypeStruct((_WROW, 128), jnp.float32),
    )(zw, b1w, k00, k10)


def kernel(x, edge_index, W1, b1, W2, b2):
    f32 = jnp.float32
    ei = edge_index.astype(jnp.int32)
    pad = jnp.full((EP - N_EDGES,), TRASH, jnp.int32)
    padd = (jnp.arange(EP - N_EDGES, dtype=jnp.int32) % (NT - N_NODES)) + TRASH
    src_p = jnp.concatenate([ei[0], pad])
    dst_p = jnp.concatenate([ei[1], padd])
    src2 = jnp.stack([src_p, src_p + NT]).reshape(NC, EP_ROWS, 128)
    dst2d = dst_p.reshape(EP_ROWS, 128)

    eye16 = jnp.eye(16, dtype=f32)
    W1f = W1.astype(f32)
    W2p = jnp.pad(W2.astype(f32), ((0, 0), (0, 1)))
    m0 = jnp.kron(eye16, W1f[:, :8])
    m1 = jnp.kron(eye16, W1f[:, 8:])
    k00 = jnp.kron(eye16, W2p[:8, :])
    k10 = jnp.kron(eye16, W2p[8:, :])
    b1w = jnp.tile(b1.astype(f32).reshape(2, 8), (1, 16))

    ones128 = jnp.ones((128,), f32)
    zerosNT = jnp.zeros((NT,), f32)
    zeros8 = jnp.zeros((NT, 8), f32)

    xr = x.reshape(N_NODES * 3 // 48, 48)

    degp = _sc_degree(dst2d, ones128, zerosNT)

    w1t = _tc_dense1(xr, m0, m1).reshape(NC * NT, 8)
    z1 = _sc_layer1(src2, dst2d, w1t, degp, zeros8)

    w2t = _tc_dense2(z1.reshape(NC, _WROW, 128), b1w, k00, k10)
    z2 = _sc_layer2(src2, dst2d, w2t.reshape(NT, 8), degp, zeros8)

    return (z2[:N_NODES, :7] + z2[NT:NT + N_NODES, :7]
            + b2.astype(f32)[None, :])


# speedup vs baseline: 1.0778x; 1.0153x over previous
import functools

import jax
import jax.numpy as jnp
from jax import lax
from jax.experimental import pallas as pl
from jax.experimental.pallas import tpu as pltpu
from jax.experimental.pallas import tpu_sc as plsc

N_NODES = 100000
N_EDGES = 3200000

NC = 2
NS = 16
NW = NC * NS

NT = 100352
TRASH = N_NODES

CHUNK = 2048
SUB = 16
NITER = 49
EPW = CHUNK * NITER
EP = EPW * NW
EP_ROWS = EP // 128
WROWS = EPW // 128
RPT = NT // NS
SCH = 784
SIT = RPT // SCH
SVR = SCH * 8 // 16

_mesh = plsc.VectorSubcoreMesh(core_axis_name="c", subcore_axis_name="s")


def _frsqrt(x):
    i = plsc.bitcast(x, jnp.int32)
    i = jnp.int32(0x5F3759DF) - lax.shift_right_arithmetic(i, 1)
    y = plsc.bitcast(i, jnp.float32)
    for _ in range(3):
        y = y * (1.5 - 0.5 * x * y * y)
    return y


@functools.partial(
    pl.kernel,
    out_type=jax.ShapeDtypeStruct((NC, NT), jnp.float32),
    mesh=_mesh,
    scratch_types=[
        pltpu.VMEM((CHUNK,), jnp.int32),
        pltpu.VMEM((CHUNK,), jnp.float32),
        pltpu.VMEM_SHARED((NT,), jnp.float32),
        pltpu.SemaphoreType.DMA,
    ],
)
def _sc_degree(dst_hbm, ones_hbm, zeros_hbm, degp_hbm, idx_v, ones_v, acc_sh, sem):
    c = lax.axis_index("c")
    s = lax.axis_index("s")
    w = s * NC + c
    base = s * RPT
    pltpu.sync_copy(zeros_hbm.at[pl.ds(base, RPT)], acc_sh.at[pl.ds(base, RPT)])
    pltpu.sync_copy(ones_hbm, ones_v)
    plsc.subcore_barrier()

    def chunk(g, carry):
        e0 = w * EPW + g * CHUNK
        pltpu.async_copy(dst_hbm.at[pl.ds(e0, CHUNK)], idx_v, sem).wait()
        pltpu.sync_copy(ones_v, acc_sh.at[idx_v], add=True)
        return carry

    lax.fori_loop(0, NITER, chunk, 0)
    plsc.subcore_barrier()
    pltpu.sync_copy(acc_sh.at[pl.ds(base, RPT)],
                    degp_hbm.at[c, pl.ds(base, RPT)])


def _make_layer(edge_split):
    if edge_split:
        ait, ch = 98, 1024
    else:
        ait, ch = 98, 2048
    NS_E = ait

    @functools.partial(
        pl.kernel,
        out_type=jax.ShapeDtypeStruct((NC * NT, 8), jnp.float32),
        mesh=_mesh,
        compiler_params=pltpu.CompilerParams(use_tc_tiling_on_sc=False,
                                             needs_layout_passes=False),
        scratch_types=[
            pltpu.VMEM((RPT,), jnp.float32),
            pltpu.VMEM((RPT,), jnp.float32),
            pltpu.VMEM((RPT,), jnp.float32),
            pltpu.VMEM((SCH, 8), jnp.float32),
            pltpu.VMEM((ch,), jnp.int32),
            pltpu.VMEM((ch,), jnp.int32),
            pltpu.VMEM((ch,), jnp.int32),
            pltpu.VMEM((ch,), jnp.int32),
            pltpu.VMEM((ch, 8), jnp.float32),
            pltpu.VMEM((ch, 8), jnp.float32),
            pltpu.VMEM_SHARED((NT, 8), jnp.float32),
            pltpu.SemaphoreType.DMA,
            pltpu.SemaphoreType.DMA,
            pltpu.SemaphoreType.DMA,
        ],
    )
    def _sc_layer(src2_hbm, dst_hbm, w_hbm, degp_hbm, zeros_hbm, z_hbm,
                  dp0, dp1, dinv_v, wbuf, sidx0, didx0, sidx1, didx1,
                  rows0, rows1, acc_sh, gsem, ssem, isem):
        c = lax.axis_index("c")
        s = lax.axis_index("s")
        t0 = s * RPT
        ji = lax.iota(jnp.int32, 16)
        rp2 = ji // 8
        colp = ji % 8

        pltpu.sync_copy(degp_hbm.at[0, pl.ds(t0, RPT)], dp0)
        pltpu.sync_copy(degp_hbm.at[1, pl.ds(t0, RPT)], dp1)

        def dinv_step(k, carry):
            sl = pl.ds(k * 16, 16)
            dinv_v[sl] = _frsqrt(dp0[sl] + dp1[sl] + 1.0)
            return carry

        lax.fori_loop(0, RPT // 16, dinv_step, 0)

        def stage_a(i, carry):
            start = t0 + i * SCH
            woff = start if edge_split else c * NT + start
            pltpu.sync_copy(w_hbm.at[pl.ds(woff, SCH)], wbuf)

            def scale(v, carry2):
                wr = rp2 + 2 * v
                w16 = plsc.load_gather(wbuf, [wr, colp])
                dv = plsc.load_gather(dinv_v, [rp2 + (i * SCH + 2 * v)])
                plsc.store_scatter(wbuf, [wr, colp], w16 * dv)
                return carry2

            lax.fori_loop(0, SVR, scale, 0)
            pltpu.sync_copy(wbuf, z_hbm.at[pl.ds(c * NT + start, SCH)])
            if edge_split:
                @pl.when(c == 0)
                def _():
                    pltpu.sync_copy(wbuf, acc_sh.at[pl.ds(start, SCH)])

                @pl.when(c != 0)
                def _():
                    pltpu.sync_copy(zeros_hbm.at[pl.ds(start, SCH)],
                                    acc_sh.at[pl.ds(start, SCH)])
            else:
                pltpu.sync_copy(wbuf, acc_sh.at[pl.ds(start, SCH)])
            return carry

        lax.fori_loop(0, SIT, stage_a, 0)
        plsc.subcore_barrier()

        def e0(k):
            if edge_split:
                return ((c * NS + s) * NS_E + k) * ch
            return (s * NS_E + k) * ch

        def load_idx(k, sbuf, dbuf):
            h1 = pltpu.async_copy(src2_hbm.at[c, pl.ds(e0(k), ch)],
                                  sbuf, isem)
            h2 = pltpu.async_copy(dst_hbm.at[pl.ds(e0(k), ch)], dbuf, isem)
            return h1, h2

        def fire_gathers(sbuf, rbuf):
            return [pltpu.async_copy(z_hbm.at[sbuf], rbuf, gsem)]

        def fire_scatters(dbuf, rbuf):
            return [pltpu.async_copy(rbuf, acc_sh.at[dbuf], ssem, add=True)]

        def drain_gathers(rbuf):
            pltpu.make_async_copy(w_hbm.at[pl.ds(0, ch)], rbuf, gsem).wait()

        h1, h2 = load_idx(0, sidx0, didx0)
        h1.wait()
        h2.wait()
        fire_gathers(sidx0, rows0)

        def pair(g, carry):
            a = 2 * g
            l1, l2 = load_idx(a + 1, sidx1, didx1)
            drain_gathers(rows0)
            sa = fire_scatters(didx0, rows0)
            l1.wait()
            l2.wait()
            gb = fire_gathers(sidx1, rows1)
            for h in sa:
                h.wait()

            @pl.when(g < ait // 2 - 1)
            def _():
                n1, n2 = load_idx(a + 2, sidx0, didx0)
                n1.wait()
                n2.wait()
                fire_gathers(sidx0, rows0)

            for h in gb:
                h.wait()
            sb_h = fire_scatters(didx1, rows1)
            for h in sb_h:
                h.wait()
            return carry

        lax.fori_loop(0, ait // 2, pair, 0)
        plsc.subcore_barrier()

        def stage_c(i, carry):
            start = t0 + i * SCH
            pltpu.sync_copy(acc_sh.at[pl.ds(start, SCH)], wbuf)

            def scale(v, carry2):
                wr = rp2 + 2 * v
                a16 = plsc.load_gather(wbuf, [wr, colp])
                dv = plsc.load_gather(dinv_v, [rp2 + (i * SCH + 2 * v)])
                plsc.store_scatter(wbuf, [wr, colp], a16 * dv)
                return carry2

            lax.fori_loop(0, SVR, scale, 0)
            pltpu.sync_copy(wbuf, z_hbm.at[pl.ds(c * NT + start, SCH)])
            return carry

        lax.fori_loop(0, SIT, stage_c, 0)

    return _sc_layer


_sc_layer1 = _make_layer(edge_split=False)
_sc_layer2 = _make_layer(edge_split=True)


_WROW = NT // 16
_TB = 448


def _dense1_body(xr_ref, m0_ref, m1_ref, outw_ref):
    xb = xr_ref[...]
    outw_ref[0] = jnp.dot(xb, m0_ref[...], preferred_element_type=jnp.float32)
    outw_ref[1] = jnp.dot(xb, m1_ref[...], preferred_element_type=jnp.float32)


def _dense2_body(zw_ref, b1w_ref, k00_ref, k10_ref, outw_ref):
    h0 = jnp.maximum(zw_ref[0] + b1w_ref[0], 0.0)
    h1 = jnp.maximum(zw_ref[1] + b1w_ref[1], 0.0)
    outw_ref[...] = (jnp.dot(h0, k00_ref[...], preferred_element_type=jnp.float32)
                     + jnp.dot(h1, k10_ref[...], preferred_element_type=jnp.float32))


def _tc_dense1(xr, m0, m1):
    return pl.pallas_call(
        _dense1_body,
        grid=(_WROW // _TB,),
        in_specs=[
            pl.BlockSpec((_TB, 48), lambda i: (i, 0)),
            pl.BlockSpec((48, 128), lambda i: (0, 0)),
            pl.BlockSpec((48, 128), lambda i: (0, 0)),
        ],
        out_specs=pl.BlockSpec((2, _TB, 128), lambda i: (0, i, 0)),
        out_shape=jax.ShapeDtypeStruct((2, _WROW, 128), jnp.float32),
    )(xr, m0, m1)


def _tc_dense2(zw, b1w, k00, k10):
    return pl.pallas_call(
        _dense2_body,
        grid=(_WROW // _TB,),
        in_specs=[
            pl.BlockSpec((2, _TB, 128), lambda i: (0, i, 0)),
            pl.BlockSpec((2, 128), lambda i: (0, 0)),
            pl.BlockSpec((128, 128), lambda i: (0, 0)),
            pl.BlockSpec((128, 128), lambda i: (0, 0)),
        ],
        out_specs=pl.BlockSpec((_TB, 128), lambda i: (i, 0)),
        out_shape=jax.ShapeDtypeStruct((_WROW, 128), jnp.float32),
    )(zw, b1w, k00, k10)


def kernel(x, edge_index, W1, b1, W2, b2):
    f32 = jnp.float32
    ei = edge_index.astype(jnp.int32)
    pad = jnp.full((EP - N_EDGES,), TRASH, jnp.int32)
    padd = (jnp.arange(EP - N_EDGES, dtype=jnp.int32) % (NT - N_NODES)) + TRASH
    src_p = jnp.concatenate([ei[0], pad])
    dst_p = jnp.concatenate([ei[1], padd])
    src2 = jnp.stack([src_p, src_p + NT])

    eye16 = jnp.eye(16, dtype=f32)
    W1f = W1.astype(f32)
    W2p = jnp.pad(W2.astype(f32), ((0, 0), (0, 1)))
    m0 = jnp.kron(eye16, W1f[:, :8])
    m1 = jnp.kron(eye16, W1f[:, 8:])
    k00 = jnp.kron(eye16, W2p[:8, :])
    k10 = jnp.kron(eye16, W2p[8:, :])
    b1w = jnp.tile(b1.astype(f32).reshape(2, 8), (1, 16))

    ones_ch = jnp.ones((CHUNK,), f32)
    zerosNT = jnp.zeros((NT,), f32)
    zeros8 = jnp.zeros((NT, 8), f32)

    xr = x.reshape(N_NODES * 3 // 48, 48)

    degp = _sc_degree(dst_p, ones_ch, zerosNT)

    w1t = _tc_dense1(xr, m0, m1).reshape(NC * NT, 8)
    z1 = _sc_layer1(src2, dst_p, w1t, degp, zeros8)

    w2t = _tc_dense2(z1.reshape(NC, _WROW, 128), b1w, k00, k10)
    z2 = _sc_layer2(src2, dst_p, w2t.reshape(NT, 8), degp, zeros8)

    return (z2[:N_NODES, :7] + z2[NT:NT + N_NODES, :7]
            + b2.astype(f32)[None, :])
